# Initial kernel scaffold; baseline (speedup 1.0000x reference)
import jax
import jax.numpy as jnp
from jax.experimental import pallas as pl


def kernel(route_codes_btr, table_ngram_2, table_ngram_3):
    raise NotImplementedError("write your pallas kernel here")


# speedup vs baseline: 1.4139x; 1.4139x over previous
import functools

import jax
import jax.numpy as jnp
from jax import lax
from jax.experimental import pallas as pl
from jax.experimental.pallas import tpu as pltpu
from jax.experimental.pallas import tpu_sc as plsc

A = 16
MEM = 64
NC, NS = 2, 16
NW = NC * NS
PAD = 8
LANES = 16


def _ngram_body(T, R, TPW, codes_hbm, table2_hbm, table3_hbm, zeros_hbm,
                out_hbm, codes_v, idx2_v, idx3_v, rows_v, sem):
    wid = lax.axis_index("s") * NC + lax.axis_index("c")
    t0 = wid * TPW

    pltpu.sync_copy(codes_hbm.at[pl.ds(t0, TPW + PAD)], codes_v)

    def idx_step(dt, carry):
        for j in range(R // LANES):
            r_vec = lax.iota(jnp.int32, LANES) + (j * LANES)
            cur = codes_v[dt + PAD, pl.ds(j * LANES, LANES)]
            prev = codes_v[dt + PAD - 1, pl.ds(j * LANES, LANES)]
            prev2 = codes_v[dt + PAD - 2, pl.ds(j * LANES, LANES)]
            idx2_v[dt, pl.ds(j * LANES, LANES)] = (
                prev + cur * A + r_vec * (A * A))
            idx3_v[dt, pl.ds(j * LANES, LANES)] = (
                prev2 + prev * A + cur * (A * A) + r_vec * (A * A * A))
        return carry

    lax.fori_loop(0, TPW, idx_step, 0)

    def gather_step(dt, carry):
        t = t0 + dt
        pltpu.async_copy(table2_hbm.at[idx2_v.at[dt]], rows_v, sem).wait()
        pltpu.sync_copy(rows_v, out_hbm.at[t, 0])
        pltpu.async_copy(table3_hbm.at[idx3_v.at[dt]], rows_v, sem).wait()
        pltpu.sync_copy(rows_v, out_hbm.at[t, 1])
        return carry

    lax.fori_loop(0, TPW, gather_step, 0)

    @pl.when(wid == 0)
    def _():
        pltpu.sync_copy(zeros_hbm, out_hbm.at[0, 0])
        pltpu.sync_copy(zeros_hbm, out_hbm.at[0, 1])
        pltpu.sync_copy(zeros_hbm, out_hbm.at[1, 1])


@functools.partial(jax.jit, static_argnames=("T", "R"))
def _ngram_lookup(codes_padded, table2, table3, zeros_row, *, T, R):
    TPW = T // NW
    mesh = plsc.VectorSubcoreMesh(core_axis_name="c", subcore_axis_name="s")
    body = functools.partial(_ngram_body, T, R, TPW)
    return pl.kernel(
        body,
        out_type=jax.ShapeDtypeStruct((T, 2, R, MEM), jnp.float32),
        mesh=mesh,
        compiler_params=pltpu.CompilerParams(use_tc_tiling_on_sc=False),
        scratch_types=[
            pltpu.VMEM((TPW + PAD, R), jnp.int32),
            pltpu.VMEM((TPW, R), jnp.int32),
            pltpu.VMEM((TPW, R), jnp.int32),
            pltpu.VMEM((R, MEM), jnp.float32),
            pltpu.SemaphoreType.DMA,
        ],
    )(codes_padded, table2, table3, zeros_row)


def kernel(route_codes_btr, table_ngram_2, table_ngram_3):
    B, S, R = route_codes_btr.shape
    codes = route_codes_btr.reshape(S, R).astype(jnp.int32)
    codes_padded = jnp.zeros((S + PAD, R), jnp.int32).at[PAD:].set(codes)
    zeros_row = jnp.zeros((R, MEM), jnp.float32)
    out = _ngram_lookup(codes_padded, table_ngram_2, table_ngram_3,
                        zeros_row, T=S, R=R)
    return out.reshape(B, S, 2 * R * MEM)


# speedup vs baseline: 1.5581x; 1.1020x over previous
import functools

import jax
import jax.numpy as jnp
from jax import lax
from jax.experimental import pallas as pl
from jax.experimental.pallas import tpu as pltpu
from jax.experimental.pallas import tpu_sc as plsc

A = 16
MEM = 64
NC, NS = 2, 16
NW = NC * NS
PAD = 8
LANES = 16


KBUF = 4


def _ngram_body(T, R, TPW, codes_hbm, table2_hbm, table3_hbm, zeros_hbm,
                out_hbm, codes_v, idx2_v, idx3_v, rows_v, *sems):
    sem_g = sems[:KBUF]
    sem_w = sems[KBUF:]
    wid = lax.axis_index("s") * NC + lax.axis_index("c")
    t0 = wid * TPW

    pltpu.sync_copy(codes_hbm.at[pl.ds(t0, TPW + PAD)], codes_v)

    def idx_step(dt, carry):
        for j in range(R // LANES):
            r_vec = lax.iota(jnp.int32, LANES) + (j * LANES)
            cur = codes_v[dt + PAD, pl.ds(j * LANES, LANES)]
            prev = codes_v[dt + PAD - 1, pl.ds(j * LANES, LANES)]
            prev2 = codes_v[dt + PAD - 2, pl.ds(j * LANES, LANES)]
            idx2_v[dt, pl.ds(j * LANES, LANES)] = (
                prev + cur * A + r_vec * (A * A))
            idx3_v[dt, pl.ds(j * LANES, LANES)] = (
                prev2 + prev * A + cur * (A * A) + r_vec * (A * A * A))
        return carry

    lax.fori_loop(0, TPW, idx_step, 0)

    def wait_gathers(b):
        pltpu.make_async_copy(out_hbm.at[0], rows_v.at[b], sem_g[b]).wait()

    def wait_write(b):
        pltpu.make_async_copy(rows_v.at[b], out_hbm.at[0], sem_w[b]).wait()

    def group_step(g, carry):
        for b in range(KBUF):
            dt = g * KBUF + b

            @pl.when(g > 0)
            def _():
                wait_write(b)

            pltpu.async_copy(table2_hbm.at[idx2_v.at[dt]],
                             rows_v.at[b, 0], sem_g[b])
            pltpu.async_copy(table3_hbm.at[idx3_v.at[dt]],
                             rows_v.at[b, 1], sem_g[b])
        for b in range(KBUF):
            dt = g * KBUF + b
            wait_gathers(b)
            pltpu.async_copy(rows_v.at[b], out_hbm.at[t0 + dt], sem_w[b])
        return carry

    lax.fori_loop(0, TPW // KBUF, group_step, 0)
    for b in range(KBUF):
        wait_write(b)

    @pl.when(wid == 0)
    def _():
        pltpu.sync_copy(zeros_hbm, out_hbm.at[0, 0])
        pltpu.sync_copy(zeros_hbm, out_hbm.at[0, 1])
        pltpu.sync_copy(zeros_hbm, out_hbm.at[1, 1])


@functools.partial(jax.jit, static_argnames=("T", "R"))
def _ngram_lookup(codes_padded, table2, table3, zeros_row, *, T, R):
    TPW = T // NW
    mesh = plsc.VectorSubcoreMesh(core_axis_name="c", subcore_axis_name="s")
    body = functools.partial(_ngram_body, T, R, TPW)
    return pl.kernel(
        body,
        out_type=jax.ShapeDtypeStruct((T, 2, R, MEM), jnp.float32),
        mesh=mesh,
        compiler_params=pltpu.CompilerParams(use_tc_tiling_on_sc=False),
        scratch_types=[
            pltpu.VMEM((TPW + PAD, R), jnp.int32),
            pltpu.VMEM((TPW, R), jnp.int32),
            pltpu.VMEM((TPW, R), jnp.int32),
            pltpu.VMEM((KBUF, 2, R, MEM), jnp.float32),
        ] + [pltpu.SemaphoreType.DMA] * (2 * KBUF),
    )(codes_padded, table2, table3, zeros_row)


def kernel(route_codes_btr, table_ngram_2, table_ngram_3):
    B, S, R = route_codes_btr.shape
    codes = route_codes_btr.reshape(S, R).astype(jnp.int32)
    codes_padded = jnp.zeros((S + PAD, R), jnp.int32).at[PAD:].set(codes)
    zeros_row = jnp.zeros((R, MEM), jnp.float32)
    out = _ngram_lookup(codes_padded, table_ngram_2, table_ngram_3,
                        zeros_row, T=S, R=R)
    return out.reshape(B, S, 2 * R * MEM)
